# Initial kernel scaffold; baseline (speedup 1.0000x reference)
import jax
import jax.numpy as jnp
from jax.experimental import pallas as pl


def kernel(x, edge_index, W1, b1, g1, be1, W2, b2, g2, be2):
    raise NotImplementedError("write your pallas kernel here")


# speedup vs baseline: 2.9649x; 2.9649x over previous
import functools

import jax
import jax.numpy as jnp
from jax import lax
from jax.experimental import pallas as pl
from jax.experimental.pallas import tpu as pltpu
from jax.experimental.pallas import tpu_sc as plsc

N = 10000
E = 320000
D = 128

L = 16
JD = D // L
NC = 2
NS = 16
NW = NC * NS
EPW = E // NW
K = 80
NCH = EPW // K
NPAD = 10240
NPS = NPAD // NS
ZR = 128

_mesh = plsc.VectorSubcoreMesh(core_axis_name="c", subcore_axis_name="s")


def _node_proj_body(x_ref, wa_ref, wb_ref, b1_ref, xa_ref, xb_ref):
    x = x_ref[...]
    dn = (((1,), (1,)), ((), ()))
    xa_ref[...] = (
        lax.dot_general(x, wa_ref[...], dn, preferred_element_type=jnp.float32)
        + b1_ref[...]
    )
    xb_ref[...] = lax.dot_general(
        x, wb_ref[...], dn, preferred_element_type=jnp.float32
    )


def _node_proj(x, w1a, w1b, b1):
    return pl.pallas_call(
        _node_proj_body,
        out_shape=(
            jax.ShapeDtypeStruct((N, D), jnp.float32),
            jax.ShapeDtypeStruct((N, D), jnp.float32),
        ),
    )(x, w1a, w1b, b1.reshape(1, D))


def _gather_body(xa_hbm, xb_hbm, row_hbm, col_hbm, h1_hbm, st_hbm,
                 ridx, cidx, ga, gb, h1s, acc, sema, semb):
    cid = lax.axis_index("c")
    sid = lax.axis_index("s")
    wid = sid * NC + cid
    base = wid * EPW

    def chunk_body(i, carry):
        s_acc, q_acc = carry
        cb = base + i * K
        pltpu.sync_copy(row_hbm.at[pl.ds(cb, K)], ridx)
        pltpu.sync_copy(col_hbm.at[pl.ds(cb, K)], cidx)
        cpa = pltpu.async_copy(xa_hbm.at[ridx], ga, sema)
        cpb = pltpu.async_copy(xb_hbm.at[cidx], gb, semb)
        cpa.wait()
        cpb.wait()

        def edge_body(e, ec):
            es, eq = ec
            ns, nq = [], []
            for j in range(JD):
                a = ga[e, pl.ds(j * L, L)]
                b = gb[e, pl.ds(j * L, L)]
                h = a + b
                h1s[e, pl.ds(j * L, L)] = h
                ns.append(es[j] + h)
                nq.append(eq[j] + h * h)
            return (tuple(ns), tuple(nq))

        s_acc, q_acc = lax.fori_loop(0, K, edge_body, (s_acc, q_acc))
        pltpu.sync_copy(h1s, h1_hbm.at[pl.ds(cb, K)])
        return (s_acc, q_acc)

    zeros = tuple(jnp.zeros((L,), jnp.float32) for _ in range(JD))
    s_acc, q_acc = lax.fori_loop(0, NCH, chunk_body, (zeros, zeros))
    for j in range(JD):
        acc[0, pl.ds(j * L, L)] = s_acc[j]
        acc[1, pl.ds(j * L, L)] = q_acc[j]
    pltpu.sync_copy(acc, st_hbm.at[wid])


_gather_pass = functools.partial(
    pl.kernel,
    out_type=(
        jax.ShapeDtypeStruct((E, D), jnp.float32),
        jax.ShapeDtypeStruct((NW, 2, D), jnp.float32),
    ),
    mesh=_mesh,
    scratch_types=[
        pltpu.VMEM((K,), jnp.int32),
        pltpu.VMEM((K,), jnp.int32),
        pltpu.VMEM((K, D), jnp.float32),
        pltpu.VMEM((K, D), jnp.float32),
        pltpu.VMEM((K, D), jnp.float32),
        pltpu.VMEM((2, D), jnp.float32),
        pltpu.SemaphoreType.DMA,
        pltpu.SemaphoreType.DMA,
    ],
)(_gather_body)


_RB = 2560
_NB = E // _RB


def _mlp2_body(h1_ref, sc1_ref, sh1_ref, w2_ref, b2_ref, h2_ref, st_ref):
    i = pl.program_id(0)
    h = jnp.maximum(h1_ref[...] * sc1_ref[...] + sh1_ref[...], 0.0)
    dn = (((1,), (1,)), ((), ()))
    h2 = (
        lax.dot_general(h, w2_ref[...], dn, preferred_element_type=jnp.float32)
        + b2_ref[...]
    )
    h2_ref[...] = h2

    @pl.when(i == 0)
    def _():
        st_ref[...] = jnp.zeros_like(st_ref)

    st_ref[0:1, :] += jnp.sum(h2, axis=0, keepdims=True)
    st_ref[1:2, :] += jnp.sum(h2 * h2, axis=0, keepdims=True)


def _mlp2(h1, scale1, shift1, W2, b2):
    return pl.pallas_call(
        _mlp2_body,
        grid=(_NB,),
        in_specs=[
            pl.BlockSpec((_RB, D), lambda i: (i, 0)),
            pl.BlockSpec((1, D), lambda i: (0, 0)),
            pl.BlockSpec((1, D), lambda i: (0, 0)),
            pl.BlockSpec((D, D), lambda i: (0, 0)),
            pl.BlockSpec((1, D), lambda i: (0, 0)),
        ],
        out_specs=(
            pl.BlockSpec((_RB, D), lambda i: (i, 0)),
            pl.BlockSpec((2, D), lambda i: (0, 0)),
        ),
        out_shape=(
            jax.ShapeDtypeStruct((E, D), jnp.float32),
            jax.ShapeDtypeStruct((2, D), jnp.float32),
        ),
    )(h1, scale1.reshape(1, D), shift1.reshape(1, D), W2, b2.reshape(1, D))


def _scatter_body(h2_hbm, row_hbm, ss_hbm, out_hbm,
                  ridx, h2v, h2r, ssv, zv, acc_s, sem):
    cid = lax.axis_index("c")
    sid = lax.axis_index("s")
    wid = sid * NC + cid
    base = wid * EPW

    pltpu.sync_copy(ss_hbm, ssv)
    sv = tuple(ssv[0, pl.ds(j * L, L)] for j in range(JD))
    tv = tuple(ssv[1, pl.ds(j * L, L)] for j in range(JD))

    def zrow(r, _):
        for j in range(JD):
            zv[r, pl.ds(j * L, L)] = jnp.zeros((L,), jnp.float32)
        return 0

    lax.fori_loop(0, ZR, zrow, 0)
    for r in range(NPS // ZR):
        pltpu.sync_copy(zv, acc_s.at[pl.ds(sid * NPS + r * ZR, ZR)])
    plsc.subcore_barrier()

    def chunk_body(i, _):
        cb = base + i * K
        pltpu.sync_copy(row_hbm.at[pl.ds(cb, K)], ridx)
        pltpu.sync_copy(h2_hbm.at[pl.ds(cb, K)], h2v)

        def edge_body(e, _2):
            for j in range(JD):
                v = h2v[e, pl.ds(j * L, L)]
                v = jnp.maximum(v * sv[j] + tv[j], 0.0)
                h2r[e, pl.ds(j * L, L)] = v
            return 0

        lax.fori_loop(0, K, edge_body, 0)
        pltpu.sync_copy(h2r, acc_s.at[ridx], add=True)
        return 0

    lax.fori_loop(0, NCH, chunk_body, 0)
    plsc.subcore_barrier()
    for r in range(NPS // ZR):
        off = sid * NPS + r * ZR
        pltpu.sync_copy(acc_s.at[pl.ds(off, ZR)], zv)
        pltpu.sync_copy(zv, out_hbm.at[cid, pl.ds(off, ZR)])


_scatter_pass = functools.partial(
    pl.kernel,
    out_type=jax.ShapeDtypeStruct((NC, NPAD, D), jnp.float32),
    mesh=_mesh,
    scratch_types=[
        pltpu.VMEM((K,), jnp.int32),
        pltpu.VMEM((K, D), jnp.float32),
        pltpu.VMEM((K, D), jnp.float32),
        pltpu.VMEM((2, D), jnp.float32),
        pltpu.VMEM((ZR, D), jnp.float32),
        pltpu.VMEM_SHARED((NPAD, D), jnp.float32),
        pltpu.SemaphoreType.DMA,
    ],
)(_scatter_body)


def _add_body(p_ref, o_ref):
    o_ref[...] = p_ref[0, :N] + p_ref[1, :N]


def _final_add(parts):
    return pl.pallas_call(
        _add_body,
        out_shape=jax.ShapeDtypeStruct((N, D), jnp.float32),
    )(parts)


def kernel(x, edge_index, W1, b1, g1, be1, W2, b2, g2, be2):
    row = edge_index[0].astype(jnp.int32)
    col = edge_index[1].astype(jnp.int32)
    w1a = W1[:, :D]
    w1b = W1[:, D:]

    xa, xb = _node_proj(x, w1a, w1b, b1)
    h1, st1 = _gather_pass(xa, xb, row, col)

    s1 = jnp.sum(st1[:, 0, :], axis=0)
    q1 = jnp.sum(st1[:, 1, :], axis=0)
    mean1 = s1 / E
    var1 = q1 / E - mean1 * mean1
    scale1 = g1 * lax.rsqrt(var1 + 1e-5)
    shift1 = be1 - mean1 * scale1

    h2, st2 = _mlp2(h1, scale1, shift1, W2, b2)

    mean2 = st2[0] / E
    var2 = st2[1] / E - mean2 * mean2
    scale2 = g2 * lax.rsqrt(var2 + 1e-5)
    shift2 = be2 - mean2 * scale2
    ss = jnp.stack([scale2, shift2])

    parts = _scatter_pass(h2, row, ss)
    return _final_add(parts)


# speedup vs baseline: 5.5507x; 1.8722x over previous
import functools

import jax
import jax.numpy as jnp
from jax import lax
from jax.experimental import pallas as pl
from jax.experimental.pallas import tpu as pltpu
from jax.experimental.pallas import tpu_sc as plsc

N = 10000
E = 320000
D = 128

L = 16
JD = D // L
NC = 2
NS = 16
NW = NC * NS
EPW = E // NW
K = 80
NCH = EPW // K
NPAD = 10240
NPS = NPAD // NS
ZR = 128

_mesh = plsc.VectorSubcoreMesh(core_axis_name="c", subcore_axis_name="s")


def _node_proj_body(x_ref, wa_ref, wb_ref, b1_ref, xa_ref, xb_ref):
    x = x_ref[...]
    dn = (((1,), (1,)), ((), ()))
    xa_ref[...] = (
        lax.dot_general(x, wa_ref[...], dn, preferred_element_type=jnp.float32)
        + b1_ref[...]
    )
    xb_ref[...] = lax.dot_general(
        x, wb_ref[...], dn, preferred_element_type=jnp.float32
    )


def _node_proj(x, w1a, w1b, b1):
    return pl.pallas_call(
        _node_proj_body,
        out_shape=(
            jax.ShapeDtypeStruct((N, D), jnp.float32),
            jax.ShapeDtypeStruct((N, D), jnp.float32),
        ),
    )(x, w1a, w1b, b1.reshape(1, D))


def _gather_body(xa_hbm, xb_hbm, row_hbm, col_hbm, h1_hbm, st_hbm,
                 ridx, cidx, ga0, ga1, ga2, gb0, gb1, gb2, hs0, hs1, hs2, acc,
                 sg0, sg1, sg2, sw0, sw1, sw2):
    cid = lax.axis_index("c")
    sid = lax.axis_index("s")
    wid = sid * NC + cid
    base = wid * EPW

    pltpu.sync_copy(row_hbm.at[pl.ds(base, EPW)], ridx)
    pltpu.sync_copy(col_hbm.at[pl.ds(base, EPW)], cidx)

    ga = (ga0, ga1, ga2)
    gb = (gb0, gb1, gb2)
    hs = (hs0, hs1, hs2)
    sg = (sg0, sg1, sg2)
    sw = (sw0, sw1, sw2)

    def start_gather(i, s):
        pltpu.async_copy(xa_hbm.at[ridx.at[pl.ds(i * K, K)]], ga[s], sg[s])
        pltpu.async_copy(xb_hbm.at[cidx.at[pl.ds(i * K, K)]], gb[s], sg[s])

    def wait_gather(s):
        pltpu.make_async_copy(xa_hbm.at[ridx.at[pl.ds(0, K)]], ga[s], sg[s]).wait()
        pltpu.make_async_copy(xb_hbm.at[cidx.at[pl.ds(0, K)]], gb[s], sg[s]).wait()

    def start_write(i, s):
        pltpu.async_copy(hs[s], h1_hbm.at[pl.ds(base + i * K, K)], sw[s])

    def wait_write(s):
        pltpu.make_async_copy(hs[s], h1_hbm.at[pl.ds(base, K)], sw[s]).wait()

    def compute(s, carry):
        def edge_body(e, ec):
            es, eq = ec
            ns, nq = [], []
            for j in range(JD):
                a = ga[s][e, pl.ds(j * L, L)]
                b = gb[s][e, pl.ds(j * L, L)]
                h = a + b
                hs[s][e, pl.ds(j * L, L)] = h
                ns.append(es[j] + h)
                nq.append(eq[j] + h * h)
            return (tuple(ns), tuple(nq))

        return lax.fori_loop(0, K, edge_body, carry)

    zeros = tuple(jnp.zeros((L,), jnp.float32) for _ in range(JD))
    carry = (zeros, zeros)

    start_gather(0, 0)
    start_gather(1, 1)
    start_gather(2, 2)
    for i in range(3):
        wait_gather(i)
        carry = compute(i, carry)
        start_write(i, i)
        start_gather(i + 3, i)

    def triple_body(q, carry):
        i0 = 3 * q
        for pos in range(3):
            i = i0 + pos
            wait_gather(pos)
            wait_write(pos)
            carry = compute(pos, carry)
            start_write(i, pos)

            @pl.when(i + 3 < NCH)
            def _():
                start_gather(i + 3, pos)

        return carry

    carry = lax.fori_loop(1, NCH // 3, triple_body, carry)
    for i in range(NCH - 2, NCH):
        pos = i % 3
        wait_gather(pos)
        wait_write(pos)
        carry = compute(pos, carry)
        start_write(i, pos)

    wait_write(2)
    wait_write(0)
    wait_write(1)

    s_acc, q_acc = carry
    for j in range(JD):
        acc[0, pl.ds(j * L, L)] = s_acc[j]
        acc[1, pl.ds(j * L, L)] = q_acc[j]
    pltpu.sync_copy(acc, st_hbm.at[wid])


_gather_pass = functools.partial(
    pl.kernel,
    out_type=(
        jax.ShapeDtypeStruct((E, D), jnp.float32),
        jax.ShapeDtypeStruct((NW, 2, D), jnp.float32),
    ),
    mesh=_mesh,
    scratch_types=(
        [pltpu.VMEM((EPW,), jnp.int32)] * 2
        + [pltpu.VMEM((K, D), jnp.float32)] * 9
        + [pltpu.VMEM((2, D), jnp.float32)]
        + [pltpu.SemaphoreType.DMA] * 6
    ),
)(_gather_body)


_RB = 2560
_NB = E // _RB


def _mlp2_body(h1_ref, sc1_ref, sh1_ref, w2_ref, b2_ref, h2_ref, st_ref):
    i = pl.program_id(0)
    h = jnp.maximum(h1_ref[...] * sc1_ref[...] + sh1_ref[...], 0.0)
    dn = (((1,), (1,)), ((), ()))
    h2 = (
        lax.dot_general(h, w2_ref[...], dn, preferred_element_type=jnp.float32)
        + b2_ref[...]
    )
    h2_ref[...] = h2

    @pl.when(i == 0)
    def _():
        st_ref[...] = jnp.zeros_like(st_ref)

    st_ref[0:1, :] += jnp.sum(h2, axis=0, keepdims=True)
    st_ref[1:2, :] += jnp.sum(h2 * h2, axis=0, keepdims=True)


def _mlp2(h1, scale1, shift1, W2, b2):
    return pl.pallas_call(
        _mlp2_body,
        grid=(_NB,),
        in_specs=[
            pl.BlockSpec((_RB, D), lambda i: (i, 0)),
            pl.BlockSpec((1, D), lambda i: (0, 0)),
            pl.BlockSpec((1, D), lambda i: (0, 0)),
            pl.BlockSpec((D, D), lambda i: (0, 0)),
            pl.BlockSpec((1, D), lambda i: (0, 0)),
        ],
        out_specs=(
            pl.BlockSpec((_RB, D), lambda i: (i, 0)),
            pl.BlockSpec((2, D), lambda i: (0, 0)),
        ),
        out_shape=(
            jax.ShapeDtypeStruct((E, D), jnp.float32),
            jax.ShapeDtypeStruct((2, D), jnp.float32),
        ),
    )(h1, scale1.reshape(1, D), shift1.reshape(1, D), W2, b2.reshape(1, D))


def _scatter_body(h2_hbm, row_hbm, ss_hbm, out_hbm,
                  ix0, ix1, ix2, ix3, v0, v1, r0, r1, ssv, acc_s,
                  sl0, sl1, sc0, sc1):
    cid = lax.axis_index("c")
    sid = lax.axis_index("s")
    wid = sid * NC + cid
    base = wid * EPW

    ix = (ix0, ix1, ix2, ix3)
    v = (v0, v1)
    r = (r0, r1)
    sl = (sl0, sl1)
    sc = (sc0, sc1)

    def start_load(i, q, s):
        pltpu.async_copy(row_hbm.at[pl.ds(base + i * K, K)], ix[q], sl[s])
        pltpu.async_copy(h2_hbm.at[pl.ds(base + i * K, K)], v[s], sl[s])

    def wait_load(s):
        pltpu.make_async_copy(row_hbm.at[pl.ds(base, K)], ix[0], sl[s]).wait()
        pltpu.make_async_copy(h2_hbm.at[pl.ds(base, K)], v[s], sl[s]).wait()

    def start_scat(q, s):
        pltpu.async_copy(r[s], acc_s.at[ix[q]], sc[s], add=True)

    def wait_scat(s):
        pltpu.make_async_copy(r[s], acc_s.at[ix[0]], sc[s]).wait()

    def compute(s, sv, tv):
        def edge_body(e, _2):
            for j in range(JD):
                h = v[s][e, pl.ds(j * L, L)]
                h = jnp.maximum(h * sv[j] + tv[j], 0.0)
                r[s][e, pl.ds(j * L, L)] = h
            return 0

        lax.fori_loop(0, K, edge_body, 0)

    start_load(0, 0, 0)
    start_load(1, 1, 1)

    pltpu.sync_copy(ss_hbm, ssv)
    sv = tuple(ssv[0, pl.ds(j * L, L)] for j in range(JD))
    tv = tuple(ssv[1, pl.ds(j * L, L)] for j in range(JD))

    def zrow(rr, _):
        for j in range(JD):
            r0[rr, pl.ds(j * L, L)] = jnp.zeros((L,), jnp.float32)
        return 0

    lax.fori_loop(0, K, zrow, 0)
    for rr in range(NPS // K):
        pltpu.sync_copy(r0, acc_s.at[pl.ds(sid * NPS + rr * K, K)])
    plsc.subcore_barrier()

    for i in range(4):
        s = i % 2
        wait_load(s)
        if i >= 2:
            wait_scat(s)
        compute(s, sv, tv)
        start_scat(i % 4, s)
        start_load(i + 2, (i + 2) % 4, s)

    def quad_body(q, _):
        i0 = 4 * q
        for pos in range(4):
            i = i0 + pos
            s = pos % 2
            wait_load(s)
            wait_scat(s)
            compute(s, sv, tv)
            start_scat(pos, s)

            @pl.when(i + 2 < NCH)
            def _():
                start_load(i + 2, (pos + 2) % 4, s)

        return 0

    lax.fori_loop(1, NCH // 4, quad_body, 0)
    wait_load(0)
    wait_scat(0)
    compute(0, sv, tv)
    start_scat(0, 0)

    wait_scat(1)
    wait_scat(0)
    plsc.subcore_barrier()
    for rr in range(NPS // K):
        off = sid * NPS + rr * K
        buf = r[rr % 2]
        pltpu.sync_copy(acc_s.at[pl.ds(off, K)], buf)
        pltpu.sync_copy(buf, out_hbm.at[cid, pl.ds(off, K)])


_scatter_pass = functools.partial(
    pl.kernel,
    out_type=jax.ShapeDtypeStruct((NC, NPAD, D), jnp.float32),
    mesh=_mesh,
    scratch_types=(
        [pltpu.VMEM((K,), jnp.int32)] * 4
        + [pltpu.VMEM((K, D), jnp.float32)] * 4
        + [pltpu.VMEM((2, D), jnp.float32)]
        + [pltpu.VMEM_SHARED((NPAD, D), jnp.float32)]
        + [pltpu.SemaphoreType.DMA] * 4
    ),
)(_scatter_body)


def _add_body(p_ref, o_ref):
    o_ref[...] = p_ref[0, :N] + p_ref[1, :N]


def _final_add(parts):
    return pl.pallas_call(
        _add_body,
        out_shape=jax.ShapeDtypeStruct((N, D), jnp.float32),
    )(parts)


def kernel(x, edge_index, W1, b1, g1, be1, W2, b2, g2, be2):
    row = edge_index[0].astype(jnp.int32)
    col = edge_index[1].astype(jnp.int32)
    w1a = W1[:, :D]
    w1b = W1[:, D:]

    xa, xb = _node_proj(x, w1a, w1b, b1)
    h1, st1 = _gather_pass(xa, xb, row, col)

    s1 = jnp.sum(st1[:, 0, :], axis=0)
    q1 = jnp.sum(st1[:, 1, :], axis=0)
    mean1 = s1 / E
    var1 = q1 / E - mean1 * mean1
    scale1 = g1 * lax.rsqrt(var1 + 1e-5)
    shift1 = be1 - mean1 * scale1

    h2, st2 = _mlp2(h1, scale1, shift1, W2, b2)

    mean2 = st2[0] / E
    var2 = st2[1] / E - mean2 * mean2
    scale2 = g2 * lax.rsqrt(var2 + 1e-5)
    shift2 = be2 - mean2 * scale2
    ss = jnp.stack([scale2, shift2])

    parts = _scatter_pass(h2, row, ss)
    return _final_add(parts)
